# baseline (speedup 1.0000x reference)
import functools

import jax
import jax.numpy as jnp
import numpy as np
from jax import lax
from jax.experimental import pallas as pl
from jax.experimental.pallas import tpu as pltpu
from jax.experimental.pallas import tpu_sc as plsc

_NUM_LEVELS = 16
_BASE_RES = 16
_LOG2_HASHMAP = 19
_B = 131072
_PRIMES = (1, 2654435761, 805459861)

_NC = 2
_NS = 16
_NW = _NC * _NS
_P = 256
_CHUNKS = _B // (_NW * _P)
_G = _P // 16

_ENC_DIMS = ((0, 1, 2), (0, 1, 3), (0, 2, 3), (1, 2, 3))


def _level_params():
    max_params = 2 ** _LOG2_HASHMAP
    off = 0
    scale, m1, m2, hashed, offs = [], [], [], [], []
    for l in range(_NUM_LEVELS):
        res = _BASE_RES * (2 ** l)
        size = min(max_params, (res + 1) ** 3)
        use_hash = (res + 1) ** 3 > size
        scale.append(float(res) - 1.0)
        if use_hash:
            m1.append(_PRIMES[1])
            m2.append(_PRIMES[2])
        else:
            m1.append(res + 1)
            m2.append((res + 1) ** 2)
        hashed.append(1 if use_hash else 0)
        offs.append(off)
        off += size
    pi = np.zeros((4, 32), np.int32)
    pi[0, :16] = np.array(m1, np.uint64).astype(np.uint32).view(np.int32)
    pi[1, :16] = np.array(m2, np.uint64).astype(np.uint32).view(np.int32)
    pi[2, :16] = np.array(hashed, np.int32)
    pi[3, :16] = np.array(offs, np.int32)
    pf = np.zeros((32,), np.float32)
    pf[:16] = np.array(scale, np.float32)
    return pi.reshape(-1), pf


_PI, _PF = _level_params()
_HMASK = (1 << _LOG2_HASHMAP) - 1


def _splat(v):
    return jnp.zeros((16,), jnp.int32) + v


def _body(coords, pi_hbm, pf_hbm, t0, t1, t2, t3, out_hbm,
          coord_v, pi_v, pf_v, idx_v, rows_v, out_v, sem):
    tables = (t0, t1, t2, t3)
    wid = lax.axis_index("s") * _NC + lax.axis_index("c")

    pltpu.sync_copy(pi_hbm, pi_v)
    pltpu.sync_copy(pf_hbm, pf_v)
    zeros16 = jnp.zeros((16,), jnp.int32)

    @pl.loop(0, _CHUNKS)
    def _chunk(ci):
        base = wid * (_CHUNKS * _P) + ci * _P
        for d in range(4):
            pltpu.sync_copy(coords.at[d, pl.ds(base, _P)],
                            coord_v.at[pl.ds(d * _P, _P)])

        for e in range(4):
            d0, d1, d2 = _ENC_DIMS[e]
            tab = tables[e]

            @pl.loop(0, _NUM_LEVELS)
            def _level(l):
                scale = jnp.zeros((16,), jnp.float32) + pf_v[pl.ds(l, 16)][0]
                m1 = zeros16 + pi_v[pl.ds(l, 16)][0]
                m2 = zeros16 + pi_v[pl.ds(32 + l, 16)][0]
                hflag = zeros16 + pi_v[pl.ds(64 + l, 16)][0]
                off = zeros16 + pi_v[pl.ds(96 + l, 16)][0]

                @pl.loop(0, _G)
                def _idx_pass(g):
                    p0 = g * 16
                    px = coord_v[pl.ds(d0 * _P + p0, 16)]
                    py = coord_v[pl.ds(d1 * _P + p0, 16)]
                    pz = coord_v[pl.ds(d2 * _P + p0, 16)]
                    gx = (px * scale + 0.5).astype(jnp.int32)
                    gy = (py * scale + 0.5).astype(jnp.int32)
                    gz = (pz * scale + 0.5).astype(jnp.int32)
                    ax = (gx, gx + 1)
                    by = (gy * m1, gy * m1 + m1)
                    cz = (gz * m2, gz * m2 + m2)
                    for c in range(8):
                        a = ax[c & 1]
                        b = by[(c >> 1) & 1]
                        cc = cz[(c >> 2) & 1]
                        hv = (a ^ b ^ cc) & _HMASK
                        dv = a + b + cc
                        row2 = (dv + (hv - dv) * hflag + off) * 2
                        idx_v[pl.ds(c * _P + p0, 16)] = row2
                        idx_v[pl.ds(8 * _P + c * _P + p0, 16)] = row2 + 1

                pltpu.async_copy(tab.at[idx_v], rows_v, sem).wait()

                col = 32 * e + 2 * l

                @pl.loop(0, _G)
                def _accum(g):
                    p0 = g * 16
                    px = coord_v[pl.ds(d0 * _P + p0, 16)]
                    py = coord_v[pl.ds(d1 * _P + p0, 16)]
                    pz = coord_v[pl.ds(d2 * _P + p0, 16)]
                    hx = px * scale + 0.5
                    hy = py * scale + 0.5
                    hz = pz * scale + 0.5
                    fx = hx - hx.astype(jnp.int32).astype(jnp.float32)
                    fy = hy - hy.astype(jnp.int32).astype(jnp.float32)
                    fz = hz - hz.astype(jnp.int32).astype(jnp.float32)
                    wx = (1.0 - fx, fx)
                    wy = (1.0 - fy, fy)
                    wz = (1.0 - fz, fz)
                    acc0 = jnp.zeros((16,), jnp.float32)
                    acc1 = jnp.zeros((16,), jnp.float32)
                    for c in range(8):
                        w = wx[c & 1] * wy[(c >> 1) & 1] * wz[(c >> 2) & 1]
                        v0 = rows_v[pl.ds(c * _P + p0, 16)]
                        v1 = rows_v[pl.ds(8 * _P + c * _P + p0, 16)]
                        acc0 = acc0 + w * v0
                        acc1 = acc1 + w * v1
                    out_v[col, pl.ds(p0, 16)] = acc0
                    out_v[col + 1, pl.ds(p0, 16)] = acc1

        pltpu.sync_copy(out_v, out_hbm.at[:, pl.ds(base, _P)])


@jax.jit
def _run(coords, pi, pf, t0, t1, t2, t3):
    mesh = plsc.VectorSubcoreMesh(core_axis_name="c", subcore_axis_name="s",
                                  num_cores=_NC, num_subcores=_NS)
    return pl.kernel(
        _body,
        out_type=jax.ShapeDtypeStruct((128, _B), jnp.float32),
        mesh=mesh,
        scratch_types=[
            pltpu.VMEM((4 * _P,), jnp.float32),
            pltpu.VMEM((128,), jnp.int32),
            pltpu.VMEM((32,), jnp.float32),
            pltpu.VMEM((16 * _P,), jnp.int32),
            pltpu.VMEM((16 * _P,), jnp.float32),
            pltpu.VMEM((128, _P), jnp.float32),
            pltpu.SemaphoreType.DMA,
        ],
    )(coords, pi, pf, t0, t1, t2, t3)


def kernel(x, t, emb_static, emb_xyt, emb_xzt, emb_yzt):
    coords = jnp.concatenate([x.T, t[None, :]], axis=0)
    pi = jnp.asarray(_PI)
    pf = jnp.asarray(_PF)
    out = _run(coords, pi, pf,
               emb_static.reshape(-1), emb_xyt.reshape(-1),
               emb_xzt.reshape(-1), emb_yzt.reshape(-1))
    return out.T


# speedup vs baseline: 1.0507x; 1.0507x over previous
import functools

import jax
import jax.numpy as jnp
import numpy as np
from jax import lax
from jax.experimental import pallas as pl
from jax.experimental.pallas import tpu as pltpu
from jax.experimental.pallas import tpu_sc as plsc

_NUM_LEVELS = 16
_BASE_RES = 16
_LOG2_HASHMAP = 19
_B = 131072
_PRIMES = (1, 2654435761, 805459861)

_NC = 2
_NS = 16
_NW = _NC * _NS
_P = 256
_CHUNKS = _B // (_NW * _P)
_G = _P // 16

_ENC_DIMS = ((0, 1, 2), (0, 1, 3), (0, 2, 3), (1, 2, 3))
_HMASK = (1 << _LOG2_HASHMAP) - 1


def _level_params():
    max_params = 2 ** _LOG2_HASHMAP
    off = 0
    scale, m1, m2, hashed, offs = [], [], [], [], []
    for l in range(_NUM_LEVELS):
        res = _BASE_RES * (2 ** l)
        size = min(max_params, (res + 1) ** 3)
        use_hash = (res + 1) ** 3 > size
        scale.append(float(res) - 1.0)
        if use_hash:
            m1.append(_PRIMES[1])
            m2.append(_PRIMES[2])
        else:
            m1.append(res + 1)
            m2.append((res + 1) ** 2)
        hashed.append(1 if use_hash else 0)
        offs.append(off)
        off += size
    pi = np.zeros((4, 32), np.int32)
    pi[0, :16] = np.array(m1, np.uint64).astype(np.uint32).view(np.int32)
    pi[1, :16] = np.array(m2, np.uint64).astype(np.uint32).view(np.int32)
    pi[2, :16] = np.array(hashed, np.int32)
    pi[3, :16] = np.array(offs, np.int32)
    pf = np.zeros((32,), np.float32)
    pf[:16] = np.array(scale, np.float32)
    return pi.reshape(-1), pf


_PI, _PF = _level_params()


def _body(coords, pi_hbm, pf_hbm, t0, t1, t2, t3, out_hbm,
          coord_v, pi_v, pf_v, idx_a, idx_b, rows_a, rows_b,
          out_v, sem_a, sem_b):
    tables = (t0, t1, t2, t3)
    wid = lax.axis_index("s") * _NC + lax.axis_index("c")

    pltpu.sync_copy(pi_hbm, pi_v)
    pltpu.sync_copy(pf_hbm, pf_v)
    zeros16 = jnp.zeros((16,), jnp.int32)

    @pl.loop(0, _CHUNKS)
    def _chunk(ci):
        base = wid * (_CHUNKS * _P) + ci * _P
        for d in range(4):
            pltpu.sync_copy(coords.at[d, pl.ds(base, _P)],
                            coord_v.at[pl.ds(d * _P, _P)])

        for e in range(4):
            d0, d1, d2 = _ENC_DIMS[e]
            tab = tables[e]

            def idx_pass(l, ibuf):
                scale = jnp.zeros((16,), jnp.float32) + pf_v[pl.ds(l, 16)][0]
                m1 = zeros16 + pi_v[pl.ds(l, 16)][0]
                m2 = zeros16 + pi_v[pl.ds(32 + l, 16)][0]
                hflag = zeros16 + pi_v[pl.ds(64 + l, 16)][0]
                off = zeros16 + pi_v[pl.ds(96 + l, 16)][0]

                @pl.loop(0, _G)
                def _ip(g):
                    p0 = g * 16
                    px = coord_v[pl.ds(d0 * _P + p0, 16)]
                    py = coord_v[pl.ds(d1 * _P + p0, 16)]
                    pz = coord_v[pl.ds(d2 * _P + p0, 16)]
                    gx = (px * scale + 0.5).astype(jnp.int32)
                    gy = (py * scale + 0.5).astype(jnp.int32)
                    gz = (pz * scale + 0.5).astype(jnp.int32)
                    ax = (gx, gx + 1)
                    by = (gy * m1, gy * m1 + m1)
                    cz = (gz * m2, gz * m2 + m2)
                    for c in range(8):
                        a = ax[c & 1]
                        b = by[(c >> 1) & 1]
                        cc = cz[(c >> 2) & 1]
                        hv = (a ^ b ^ cc) & _HMASK
                        dv = a + b + cc
                        row2 = (dv + (hv - dv) * hflag + off) * 2
                        ibuf[pl.ds(c * _P + p0, 16)] = row2
                        ibuf[pl.ds(8 * _P + c * _P + p0, 16)] = row2 + 1

            def fire(ibuf, fbuf, sem):
                pltpu.async_copy(tab.at[ibuf], fbuf, sem)

            def drain(ibuf, fbuf, sem):
                pltpu.make_async_copy(tab.at[ibuf], fbuf, sem).wait()

            def accum(l, fbuf):
                scale = jnp.zeros((16,), jnp.float32) + pf_v[pl.ds(l, 16)][0]
                col = 32 * e + 2 * l

                @pl.loop(0, _G)
                def _ap(g):
                    p0 = g * 16
                    px = coord_v[pl.ds(d0 * _P + p0, 16)]
                    py = coord_v[pl.ds(d1 * _P + p0, 16)]
                    pz = coord_v[pl.ds(d2 * _P + p0, 16)]
                    hx = px * scale + 0.5
                    hy = py * scale + 0.5
                    hz = pz * scale + 0.5
                    fx = hx - hx.astype(jnp.int32).astype(jnp.float32)
                    fy = hy - hy.astype(jnp.int32).astype(jnp.float32)
                    fz = hz - hz.astype(jnp.int32).astype(jnp.float32)
                    wx = (1.0 - fx, fx)
                    wy = (1.0 - fy, fy)
                    wz = (1.0 - fz, fz)
                    acc0 = jnp.zeros((16,), jnp.float32)
                    acc1 = jnp.zeros((16,), jnp.float32)
                    for c in range(8):
                        w = wx[c & 1] * wy[(c >> 1) & 1] * wz[(c >> 2) & 1]
                        v0 = fbuf[pl.ds(c * _P + p0, 16)]
                        v1 = fbuf[pl.ds(8 * _P + c * _P + p0, 16)]
                        acc0 = acc0 + w * v0
                        acc1 = acc1 + w * v1
                    out_v[col, pl.ds(p0, 16)] = acc0
                    out_v[col + 1, pl.ds(p0, 16)] = acc1

            idx_pass(0, idx_a)
            fire(idx_a, rows_a, sem_a)

            @pl.loop(1, 15, step=2)
            def _pipe(l):
                idx_pass(l, idx_b)
                fire(idx_b, rows_b, sem_b)
                drain(idx_a, rows_a, sem_a)
                accum(l - 1, rows_a)
                idx_pass(l + 1, idx_a)
                fire(idx_a, rows_a, sem_a)
                drain(idx_b, rows_b, sem_b)
                accum(l, rows_b)

            idx_pass(15, idx_b)
            fire(idx_b, rows_b, sem_b)
            drain(idx_a, rows_a, sem_a)
            accum(14, rows_a)
            drain(idx_b, rows_b, sem_b)
            accum(15, rows_b)

        pltpu.sync_copy(out_v, out_hbm.at[:, pl.ds(base, _P)])


@jax.jit
def _run(coords, pi, pf, t0, t1, t2, t3):
    mesh = plsc.VectorSubcoreMesh(core_axis_name="c", subcore_axis_name="s",
                                  num_cores=_NC, num_subcores=_NS)
    return pl.kernel(
        _body,
        out_type=jax.ShapeDtypeStruct((128, _B), jnp.float32),
        mesh=mesh,
        scratch_types=[
            pltpu.VMEM((4 * _P,), jnp.float32),
            pltpu.VMEM((128,), jnp.int32),
            pltpu.VMEM((32,), jnp.float32),
            pltpu.VMEM((16 * _P,), jnp.int32),
            pltpu.VMEM((16 * _P,), jnp.int32),
            pltpu.VMEM((16 * _P,), jnp.float32),
            pltpu.VMEM((16 * _P,), jnp.float32),
            pltpu.VMEM((128, _P), jnp.float32),
            pltpu.SemaphoreType.DMA,
            pltpu.SemaphoreType.DMA,
        ],
    )(coords, pi, pf, t0, t1, t2, t3)


def kernel(x, t, emb_static, emb_xyt, emb_xzt, emb_yzt):
    coords = jnp.concatenate([x.T, t[None, :]], axis=0)
    pi = jnp.asarray(_PI)
    pf = jnp.asarray(_PF)
    out = _run(coords, pi, pf,
               emb_static.reshape(-1), emb_xyt.reshape(-1),
               emb_xzt.reshape(-1), emb_yzt.reshape(-1))
    return out.T


# speedup vs baseline: 5.1669x; 4.9176x over previous
import functools

import jax
import jax.numpy as jnp
import numpy as np
from jax import lax
from jax.experimental import pallas as pl
from jax.experimental.pallas import tpu as pltpu
from jax.experimental.pallas import tpu_sc as plsc

_NUM_LEVELS = 16
_BASE_RES = 16
_LOG2_HASHMAP = 19
_B = 131072
_PRIMES = (1, 2654435761, 805459861)

_NC = 2
_NS = 16
_NW = _NC * _NS
_P = 256
_CHUNKS = _B // (_NW * _P)
_G = _P // 16

_ENC_DIMS = ((0, 1, 2), (0, 1, 3), (0, 2, 3), (1, 2, 3))
_HMASK = (1 << _LOG2_HASHMAP) - 1


def _level_params():
    max_params = 2 ** _LOG2_HASHMAP
    off = 0
    scale, m1, m2, hashed, offs = [], [], [], [], []
    for l in range(_NUM_LEVELS):
        res = _BASE_RES * (2 ** l)
        size = min(max_params, (res + 1) ** 3)
        use_hash = (res + 1) ** 3 > size
        scale.append(float(res) - 1.0)
        if use_hash:
            m1.append(_PRIMES[1])
            m2.append(_PRIMES[2])
        else:
            m1.append(res + 1)
            m2.append((res + 1) ** 2)
        hashed.append(1 if use_hash else 0)
        offs.append(off)
        off += size
    pi = np.zeros((4, 32), np.int32)
    pi[0, :16] = np.array(m1, np.uint64).astype(np.uint32).view(np.int32)
    pi[1, :16] = np.array(m2, np.uint64).astype(np.uint32).view(np.int32)
    pi[2, :16] = np.array(hashed, np.int32)
    pi[3, :16] = np.array(offs, np.int32)
    pf = np.zeros((32,), np.float32)
    pf[:16] = np.array(scale, np.float32)
    return pi.reshape(-1), pf


_PI, _PF = _level_params()


def _body(coords, pi_hbm, pf_hbm, t0a, t0b, t1a, t1b, t2a, t2b, t3a, t3b,
          out_hbm, coord_v, pi_v, pf_v, idx_a, idx_b, rows_a, rows_b,
          out_v, sem_a, sem_b):
    tables = ((t0a, t0b), (t1a, t1b), (t2a, t2b), (t3a, t3b))
    wid = lax.axis_index("s") * _NC + lax.axis_index("c")

    pltpu.sync_copy(pi_hbm, pi_v)
    pltpu.sync_copy(pf_hbm, pf_v)
    zeros16 = jnp.zeros((16,), jnp.int32)

    @pl.loop(0, _CHUNKS)
    def _chunk(ci):
        base = wid * (_CHUNKS * _P) + ci * _P
        for d in range(4):
            pltpu.sync_copy(coords.at[d, pl.ds(base, _P)],
                            coord_v.at[pl.ds(d * _P, _P)])

        for e in range(4):
            d0, d1, d2 = _ENC_DIMS[e]
            tabf0, tabf1 = tables[e]

            def idx_pass(l, ibuf):
                scale = jnp.zeros((16,), jnp.float32) + pf_v[pl.ds(l, 16)][0]
                m1 = zeros16 + pi_v[pl.ds(l, 16)][0]
                m2 = zeros16 + pi_v[pl.ds(32 + l, 16)][0]
                hflag = zeros16 + pi_v[pl.ds(64 + l, 16)][0]
                off = zeros16 + pi_v[pl.ds(96 + l, 16)][0]

                @pl.loop(0, _G)
                def _ip(g):
                    p0 = g * 16
                    px = coord_v[pl.ds(d0 * _P + p0, 16)]
                    py = coord_v[pl.ds(d1 * _P + p0, 16)]
                    pz = coord_v[pl.ds(d2 * _P + p0, 16)]
                    gx = (px * scale + 0.5).astype(jnp.int32)
                    gy = (py * scale + 0.5).astype(jnp.int32)
                    gz = (pz * scale + 0.5).astype(jnp.int32)
                    ax = (gx, gx + 1)
                    by = (gy * m1, gy * m1 + m1)
                    cz = (gz * m2, gz * m2 + m2)
                    for c in range(8):
                        a = ax[c & 1]
                        b = by[(c >> 1) & 1]
                        cc = cz[(c >> 2) & 1]
                        hv = (a ^ b ^ cc) & _HMASK
                        dv = a + b + cc
                        row = dv + (hv - dv) * hflag + off
                        ibuf[pl.ds(c * _P + p0, 16)] = row

            def fire(ibuf, fbuf, sem):
                pltpu.async_copy(tabf0.at[ibuf], fbuf.at[pl.ds(0, 8 * _P)],
                                 sem)
                pltpu.async_copy(tabf1.at[ibuf],
                                 fbuf.at[pl.ds(8 * _P, 8 * _P)], sem)

            def drain(ibuf, fbuf, sem):
                pltpu.make_async_copy(tabf0.at[ibuf],
                                      fbuf.at[pl.ds(0, 8 * _P)], sem).wait()
                pltpu.make_async_copy(tabf1.at[ibuf],
                                      fbuf.at[pl.ds(8 * _P, 8 * _P)],
                                      sem).wait()

            def accum(l, fbuf):
                scale = jnp.zeros((16,), jnp.float32) + pf_v[pl.ds(l, 16)][0]
                col = 32 * e + 2 * l

                @pl.loop(0, _G)
                def _ap(g):
                    p0 = g * 16
                    px = coord_v[pl.ds(d0 * _P + p0, 16)]
                    py = coord_v[pl.ds(d1 * _P + p0, 16)]
                    pz = coord_v[pl.ds(d2 * _P + p0, 16)]
                    hx = px * scale + 0.5
                    hy = py * scale + 0.5
                    hz = pz * scale + 0.5
                    fx = hx - hx.astype(jnp.int32).astype(jnp.float32)
                    fy = hy - hy.astype(jnp.int32).astype(jnp.float32)
                    fz = hz - hz.astype(jnp.int32).astype(jnp.float32)
                    wx = (1.0 - fx, fx)
                    wy = (1.0 - fy, fy)
                    wz = (1.0 - fz, fz)
                    acc0 = jnp.zeros((16,), jnp.float32)
                    acc1 = jnp.zeros((16,), jnp.float32)
                    for c in range(8):
                        w = wx[c & 1] * wy[(c >> 1) & 1] * wz[(c >> 2) & 1]
                        v0 = fbuf[pl.ds(c * _P + p0, 16)]
                        v1 = fbuf[pl.ds(8 * _P + c * _P + p0, 16)]
                        acc0 = acc0 + w * v0
                        acc1 = acc1 + w * v1
                    out_v[col, pl.ds(p0, 16)] = acc0
                    out_v[col + 1, pl.ds(p0, 16)] = acc1

            idx_pass(0, idx_a)
            fire(idx_a, rows_a, sem_a)

            @pl.loop(1, 15, step=2)
            def _pipe(l):
                idx_pass(l, idx_b)
                fire(idx_b, rows_b, sem_b)
                drain(idx_a, rows_a, sem_a)
                accum(l - 1, rows_a)
                idx_pass(l + 1, idx_a)
                fire(idx_a, rows_a, sem_a)
                drain(idx_b, rows_b, sem_b)
                accum(l, rows_b)

            idx_pass(15, idx_b)
            fire(idx_b, rows_b, sem_b)
            drain(idx_a, rows_a, sem_a)
            accum(14, rows_a)
            drain(idx_b, rows_b, sem_b)
            accum(15, rows_b)

        pltpu.sync_copy(out_v, out_hbm.at[:, pl.ds(base, _P)])


@jax.jit
def _run(coords, pi, pf, t0a, t0b, t1a, t1b, t2a, t2b, t3a, t3b):
    mesh = plsc.VectorSubcoreMesh(core_axis_name="c", subcore_axis_name="s",
                                  num_cores=_NC, num_subcores=_NS)
    return pl.kernel(
        _body,
        out_type=jax.ShapeDtypeStruct((128, _B), jnp.float32),
        mesh=mesh,
        scratch_types=[
            pltpu.VMEM((4 * _P,), jnp.float32),
            pltpu.VMEM((128,), jnp.int32),
            pltpu.VMEM((32,), jnp.float32),
            pltpu.VMEM((8 * _P,), jnp.int32),
            pltpu.VMEM((8 * _P,), jnp.int32),
            pltpu.VMEM((16 * _P,), jnp.float32),
            pltpu.VMEM((16 * _P,), jnp.float32),
            pltpu.VMEM((128, _P), jnp.float32),
            pltpu.SemaphoreType.DMA,
            pltpu.SemaphoreType.DMA,
        ],
    )(coords, pi, pf, t0a, t0b, t1a, t1b, t2a, t2b, t3a, t3b)


def kernel(x, t, emb_static, emb_xyt, emb_xzt, emb_yzt):
    coords = jnp.concatenate([x.T, t[None, :]], axis=0)
    pi = jnp.asarray(_PI)
    pf = jnp.asarray(_PF)
    out = _run(coords, pi, pf,
               emb_static[:, 0], emb_static[:, 1],
               emb_xyt[:, 0], emb_xyt[:, 1],
               emb_xzt[:, 0], emb_xzt[:, 1],
               emb_yzt[:, 0], emb_yzt[:, 1])
    return out.T
